# Initial kernel scaffold; baseline (speedup 1.0000x reference)
import jax
import jax.numpy as jnp
from jax.experimental import pallas as pl


def kernel(x, edge_attr, atom_tables, bond_tables):
    raise NotImplementedError("write your pallas kernel here")


# speedup vs baseline: 11.5887x; 11.5887x over previous
import functools

import jax
import jax.numpy as jnp
from jax.experimental import pallas as pl

_DIM = 128
_ATOM_DIMS = (119, 4, 12, 12, 10, 6, 6, 2, 2)
_BOND_DIMS = (5, 6, 2)


def _embed_block(idx_ref, tab_ref, out_ref, *, offsets, vpad):
    b = idx_ref.shape[0]
    lanes = jax.lax.broadcasted_iota(jnp.int32, (b, vpad), 1)
    mh = jnp.zeros((b, vpad), dtype=jnp.float32)
    for f, off in enumerate(offsets):
        idx = idx_ref[:, f][:, None] + off
        mh = mh + (lanes == idx).astype(jnp.float32)
    out_ref[...] = jnp.dot(mh, tab_ref[...], preferred_element_type=jnp.float32)


def _embed_sum(idx, tables, feat_dims, block_rows):
    n, nf = idx.shape
    offsets = []
    off = 0
    for v in feat_dims:
        offsets.append(off)
        off += v
    vocab = off
    vpad = ((vocab + 127) // 128) * 128
    tab = jnp.zeros((vpad, _DIM), dtype=jnp.float32)
    r = 0
    for t, v in zip(tables, feat_dims):
        tab = jax.lax.dynamic_update_slice(tab, t, (r, 0))
        r += v
    grid = (n + block_rows - 1) // block_rows
    body = functools.partial(_embed_block, offsets=tuple(offsets), vpad=vpad)
    return pl.pallas_call(
        body,
        grid=(grid,),
        in_specs=[
            pl.BlockSpec((block_rows, nf), lambda i: (i, 0)),
            pl.BlockSpec((vpad, _DIM), lambda i: (0, 0)),
        ],
        out_specs=pl.BlockSpec((block_rows, _DIM), lambda i: (i, 0)),
        out_shape=jax.ShapeDtypeStruct((n, _DIM), jnp.float32),
    )(idx, tab)


def kernel(x, edge_attr, atom_tables, bond_tables):
    x_emb = _embed_sum(x, atom_tables, _ATOM_DIMS, 2000)
    e_emb = _embed_sum(edge_attr, bond_tables, _BOND_DIMS, 2048)
    return (x_emb, e_emb)
